# baseline (speedup 1.0000x reference)
import functools

import jax
import jax.numpy as jnp
from jax import lax
from jax.experimental import pallas as pl
from jax.experimental.pallas import tpu as pltpu
from jax.experimental.pallas import tpu_sc as plsc

_NC = 2
_NS = 16
_NW = _NC * _NS
_CHUNK = 128


def _zdiv(x):
    for d in range(min(128, x), 0, -1):
        if x % d == 0:
            return d
    return 1


def _make_sc_scatter(h, dw, nchunks, chp):
    base_nj = nchunks // _NW
    extra = nchunks % _NW
    njmax = base_nj + (1 if extra else 0)
    rows_per_tile = h // _NS
    zr = _zdiv(rows_per_tile)
    mesh = plsc.VectorSubcoreMesh(
        core_axis_name="c", subcore_axis_name="s",
        num_cores=_NC, num_subcores=_NS)

    @functools.partial(
        pl.kernel,
        out_type=jax.ShapeDtypeStruct((_NC, 2, h, dw), jnp.float32),
        mesh=mesh,
        scratch_types=[
            pltpu.VMEM_SHARED((h, dw), jnp.float32),
            pltpu.VMEM((njmax, _CHUNK), jnp.int32),
            pltpu.VMEM((njmax, _CHUNK), jnp.int32),
            pltpu.VMEM((4, _CHUNK, dw), jnp.float32),
            pltpu.VMEM((zr, dw), jnp.float32),
            pltpu.SemaphoreType.DMA((2,)),
            pltpu.SemaphoreType.DMA((4,)),
            pltpu.SemaphoreType.DMA((4,)),
        ],
        compiler_params=pltpu.CompilerParams(use_tc_tiling_on_sc=False),
    )
    def sc_kernel(y0_hbm, y1_hbm, ei0_hbm, ei1_hbm,
                  out_hbm, sh, src_v, dst_v, rows_v, zbuf,
                  isem, gsem, ssem):
        c = lax.axis_index("c")
        s = lax.axis_index("s")
        w = s * _NC + c
        r0 = s * rows_per_tile
        nj = base_nj + jnp.where(w < extra, 1, 0)
        lo = w * base_nj + jnp.minimum(w, extra)

        def zero_body(i, carry):
            for k in range(dw // 16):
                zbuf[i, pl.ds(k * 16, 16)] = jnp.zeros((16,), jnp.float32)
            return carry
        lax.fori_loop(0, zr, zero_body, 0)

        for r, (ytab, ei_hbm) in enumerate(((y0_hbm, ei0_hbm),
                                            (y1_hbm, ei1_hbm))):
            pltpu.async_copy(ei_hbm.at[pl.ds(lo, njmax)], src_v,
                             isem.at[0])
            pltpu.async_copy(ei_hbm.at[pl.ds(chp + lo, njmax)], dst_v,
                             isem.at[1])
            for k in range(rows_per_tile // zr):
                pltpu.sync_copy(zbuf, sh.at[pl.ds(r0 + k * zr, zr)])
            pltpu.make_async_copy(ei_hbm.at[pl.ds(0, njmax)], src_v,
                                  isem.at[0]).wait()
            pltpu.make_async_copy(ei_hbm.at[pl.ds(0, njmax)], dst_v,
                                  isem.at[1]).wait()
            plsc.subcore_barrier()

            def issue_gather(j):
                b = jnp.bitwise_and(j, 3)
                pltpu.async_copy(ytab.at[src_v.at[j]], rows_v.at[b],
                                 gsem.at[b])

            def wait_gather(j):
                b = jnp.bitwise_and(j, 3)
                pltpu.make_async_copy(ytab.at[src_v.at[j]], rows_v.at[b],
                                      gsem.at[b]).wait()

            def issue_scatter(j):
                b = jnp.bitwise_and(j, 3)
                pltpu.async_copy(rows_v.at[b], sh.at[dst_v.at[j]],
                                 ssem.at[b], add=True)

            def wait_scatter(j):
                b = jnp.bitwise_and(j, 3)
                pltpu.make_async_copy(rows_v.at[b], sh.at[dst_v.at[j]],
                                      ssem.at[b]).wait()

            @pl.when(nj >= 1)
            def _():
                issue_gather(jnp.int32(0))

            @pl.when(nj >= 2)
            def _():
                issue_gather(jnp.int32(1))

            @pl.when(nj >= 1)
            def _():
                wait_gather(jnp.int32(0))
                issue_scatter(jnp.int32(0))

            @pl.when(nj >= 3)
            def _():
                issue_gather(jnp.int32(2))

            @pl.when(nj >= 2)
            def _():
                wait_gather(jnp.int32(1))
                issue_scatter(jnp.int32(1))

            @pl.when(nj >= 4)
            def _():
                issue_gather(jnp.int32(3))

            def body(j, carry):
                wait_gather(j)
                issue_scatter(j)
                wait_scatter(j - 2)
                issue_gather(j + 2)
                return carry

            lax.fori_loop(2, jnp.maximum(nj - 2, 2), body, 0)

            @pl.when(nj >= 4)
            def _():
                wait_gather(nj - 2)
                issue_scatter(nj - 2)
                wait_scatter(nj - 4)

            @pl.when(nj >= 3)
            def _():
                wait_gather(nj - 1)
                issue_scatter(nj - 1)
                wait_scatter(nj - 3)

            @pl.when(nj >= 2)
            def _():
                wait_scatter(nj - 2)

            @pl.when(nj >= 1)
            def _():
                wait_scatter(nj - 1)

            plsc.subcore_barrier()
            pltpu.sync_copy(sh.at[pl.ds(r0, rows_per_tile)],
                            out_hbm.at[c, r, pl.ds(r0, rows_per_tile)])

    return sc_kernel


def _tc1(x, w0a, w1a, wl, bn):
    h = x.shape[0]
    d_in = x.shape[1]
    dwa = w0a.shape[1]
    dh = wl.shape[1]

    def body(x_ref, w0_ref, w1_ref, wl_ref, y0_ref, y1_ref, xl_ref):
        xb = x_ref[...]
        ones = (lax.broadcasted_iota(jnp.int32, (bn, dwa), 1)
                >= dh).astype(jnp.float32)
        y0_ref[...] = jnp.dot(xb, w0_ref[...],
                              preferred_element_type=jnp.float32) + ones
        y1_ref[...] = jnp.dot(xb, w1_ref[...],
                              preferred_element_type=jnp.float32) + ones
        xl_ref[...] = jnp.dot(xb, wl_ref[...],
                              preferred_element_type=jnp.float32)

    return pl.pallas_call(
        body,
        grid=(h // bn,),
        in_specs=[
            pl.BlockSpec((bn, d_in), lambda i: (i, 0)),
            pl.BlockSpec((d_in, dwa), lambda i: (0, 0)),
            pl.BlockSpec((d_in, dwa), lambda i: (0, 0)),
            pl.BlockSpec((d_in, dh), lambda i: (0, 0)),
        ],
        out_specs=[
            pl.BlockSpec((bn, dwa), lambda i: (i, 0)),
            pl.BlockSpec((bn, dwa), lambda i: (i, 0)),
            pl.BlockSpec((bn, dh), lambda i: (i, 0)),
        ],
        out_shape=[
            jax.ShapeDtypeStruct((h, dwa), jnp.float32),
            jax.ShapeDtypeStruct((h, dwa), jnp.float32),
            jax.ShapeDtypeStruct((h, dh), jnp.float32),
        ],
    )(x, w0a, w1a, wl)


def _tc2(p1, xl, b1, w20, w21, w2l, bn):
    h = xl.shape[0]
    dwa = p1.shape[3]
    dh = xl.shape[1]
    do = w20.shape[1]
    rep = dh // (dwa - dh)

    def body(p_ref, xl_ref, b1_ref, w20_ref, w21_ref, w2l_ref,
             y0_ref, y1_ref, hl_ref, invd_ref):
        a0 = p_ref[0, 0] + p_ref[1, 0]
        a1 = p_ref[0, 1] + p_ref[1, 1]
        invd0 = 1.0 / jnp.maximum(a0[:, dh:], 1.0)
        invd1 = 1.0 / jnp.maximum(a1[:, dh:], 1.0)
        s0 = a0[:, :dh] * jnp.concatenate([invd0] * rep, axis=1)
        s1 = a1[:, :dh] * jnp.concatenate([invd1] * rep, axis=1)
        hb = jnp.maximum(s0 + s1 + xl_ref[...] + b1_ref[...], 0.0)
        y0_ref[...] = jnp.dot(hb, w20_ref[...],
                              preferred_element_type=jnp.float32)
        y1_ref[...] = jnp.dot(hb, w21_ref[...],
                              preferred_element_type=jnp.float32)
        hl_ref[...] = jnp.dot(hb, w2l_ref[...],
                              preferred_element_type=jnp.float32)
        invd_ref[...] = jnp.concatenate([invd0[:, :do], invd1[:, :do]],
                                        axis=1)

    return pl.pallas_call(
        body,
        grid=(h // bn,),
        in_specs=[
            pl.BlockSpec((_NC, 2, bn, dwa), lambda i: (0, 0, i, 0)),
            pl.BlockSpec((bn, dh), lambda i: (i, 0)),
            pl.BlockSpec((1, dh), lambda i: (0, 0)),
            pl.BlockSpec((dh, do), lambda i: (0, 0)),
            pl.BlockSpec((dh, do), lambda i: (0, 0)),
            pl.BlockSpec((dh, do), lambda i: (0, 0)),
        ],
        out_specs=[
            pl.BlockSpec((bn, do), lambda i: (i, 0)),
            pl.BlockSpec((bn, do), lambda i: (i, 0)),
            pl.BlockSpec((bn, do), lambda i: (i, 0)),
            pl.BlockSpec((bn, 2 * do), lambda i: (i, 0)),
        ],
        out_shape=[
            jax.ShapeDtypeStruct((h, do), jnp.float32),
            jax.ShapeDtypeStruct((h, do), jnp.float32),
            jax.ShapeDtypeStruct((h, do), jnp.float32),
            jax.ShapeDtypeStruct((h, 2 * do), jnp.float32),
        ],
    )(p1, xl, b1, w20, w21, w2l)


def _tc3(p2, hl, invd, b2, bn):
    h, do = hl.shape

    def body(p_ref, hl_ref, invd_ref, b2_ref, o_ref):
        a0 = p_ref[0, 0] + p_ref[1, 0]
        a1 = p_ref[0, 1] + p_ref[1, 1]
        o_ref[...] = (a0 * invd_ref[:, :do] + a1 * invd_ref[:, do:]
                      + hl_ref[...] + b2_ref[...])

    return pl.pallas_call(
        body,
        grid=(h // bn,),
        in_specs=[
            pl.BlockSpec((_NC, 2, bn, do), lambda i: (0, 0, i, 0)),
            pl.BlockSpec((bn, do), lambda i: (i, 0)),
            pl.BlockSpec((bn, 2 * do), lambda i: (i, 0)),
            pl.BlockSpec((1, do), lambda i: (0, 0)),
        ],
        out_specs=pl.BlockSpec((bn, do), lambda i: (i, 0)),
        out_shape=jax.ShapeDtypeStruct((h, do), jnp.float32),
    )(p2, hl, invd, b2)


def kernel(x, edge_index0, edge_index1, W1_0, W1_1, W1_loop, b1,
           W2_0, W2_1, W2_loop, b2):
    n, d_in = x.shape
    dh = W1_0.shape[1]
    do = W2_0.shape[1]
    e = edge_index0.shape[1]
    nd = 16
    dwa = dh + nd
    need_dummy = bool(e % _CHUNK)
    h = -(-(n + (1 if need_dummy else 0)) // 80) * 80
    bn = h // 10
    if h != n:
        x = jnp.pad(x, ((0, h - n), (0, 0)))

    w0a = jnp.pad(W1_0, ((0, 0), (0, nd)))
    w1a = jnp.pad(W1_1, ((0, 0), (0, nd)))

    e_pad = -(-e // _CHUNK) * _CHUNK
    nchunks = e_pad // _CHUNK
    chp = _NW * (-(-nchunks // _NW))

    def prep(ei):
        ei = ei.astype(jnp.int32)
        if e_pad != e:
            fill = jnp.stack([jnp.zeros((e_pad - e,), jnp.int32),
                              jnp.full((e_pad - e,), n, jnp.int32)])
            ei = jnp.concatenate([ei, fill], axis=1)
        return jnp.pad(ei.reshape(2, nchunks, _CHUNK),
                       ((0, 0), (0, chp - nchunks), (0, 0))
                       ).reshape(2 * chp, _CHUNK)

    ei0 = prep(edge_index0)
    ei1 = prep(edge_index1)

    y0a, y1a, xl = _tc1(x, w0a, w1a, W1_loop, bn)
    p1 = _make_sc_scatter(h, dwa, nchunks, chp)(y0a, y1a, ei0, ei1)
    y20, y21, hl, invd = _tc2(p1, xl, b1.reshape(1, dh),
                              W2_0, W2_1, W2_loop, bn)
    p2 = _make_sc_scatter(h, do, nchunks, chp)(y20, y21, ei0, ei1)
    o = _tc3(p2, hl, invd, b2.reshape(1, do), bn)
    return o[:n] if h != n else o
